# baseline (speedup 1.0000x reference)
import functools

import jax
import jax.numpy as jnp
from jax import lax
from jax.experimental import pallas as pl
from jax.experimental.pallas import tpu as pltpu
from jax.experimental.pallas import tpu_sc as plsc

N = 10000
D = 128
NPAD = 10240
E = 320000
NC, NS, L = 2, 16, 16
NW = NC * NS
EPT = 10240
EPAD = EPT * NW
RPS = NPAD // NS
CH = 128
K = EPT // CH
NCHUNK = NW * K
FAST = 1
KF = 160
KS = 0
PK = 32
PK2 = PK // 2
NPHF = KF // PK

_mesh = plsc.VectorSubcoreMesh(core_axis_name="c", subcore_axis_name="s")


@functools.partial(
    pl.kernel,
    out_type=jax.ShapeDtypeStruct((NC, NPAD, D), jnp.float32),
    mesh=_mesh,
    scratch_types=[
        pltpu.VMEM_SHARED((NPAD, D), jnp.float32),
        pltpu.VMEM((CH, D), jnp.float32),
        pltpu.VMEM((K, CH), jnp.int32),
        pltpu.VMEM((64, D), jnp.float32),
        pltpu.SemaphoreType.DMA,
    ],
)
def _deg_kernel(dst_hbm, hist_hbm, shared_h, ones_v, didx, zbuf, sem):
    cid = lax.axis_index("c")
    sid = lax.axis_index("s")
    wid = sid * NC + cid

    def fill(i, _):
        for j in range(D // L):
            zbuf[i, pl.ds(j * L, L)] = jnp.zeros((L,), jnp.float32)
        return 0

    lax.fori_loop(0, 64, fill, 0)

    def fill1(i, _):
        for j in range(D // L):
            ones_v[i, pl.ds(j * L, L)] = jnp.ones((L,), jnp.float32)
        return 0

    lax.fori_loop(0, CH, fill1, 0)
    for j in range(RPS // 64):
        pltpu.sync_copy(zbuf, shared_h.at[pl.ds(sid * RPS + j * 64, 64)])
    pltpu.sync_copy(dst_hbm.at[pl.ds(wid * K, K)], didx)
    plsc.subcore_barrier()

    def grp(g, _):
        k = g * 8
        for j in range(8):
            pltpu.async_copy(ones_v, shared_h.at[didx.at[k + j]], sem,
                             add=True)
        for j in range(8):
            pltpu.make_async_copy(ones_v, shared_h.at[didx.at[k + j]],
                                  sem).wait()
        return 0

    lax.fori_loop(0, K // 8, grp, 0)
    plsc.subcore_barrier()
    pltpu.sync_copy(
        shared_h.at[pl.ds(sid * RPS, RPS)],
        hist_hbm.at[cid, pl.ds(sid * RPS, RPS)],
    )


@functools.partial(
    pl.kernel,
    out_type=jax.ShapeDtypeStruct((NC, NPAD, D), jnp.float32),
    mesh=_mesh,
    scratch_types=[
        pltpu.VMEM_SHARED((NPAD, D), jnp.float32),
        pltpu.VMEM((CH, D), jnp.float32),
        pltpu.VMEM((CH, D), jnp.float32),
        pltpu.VMEM((PK, CH), jnp.int32),
        pltpu.VMEM((PK, CH), jnp.int32),
        pltpu.SemaphoreType.DMA,
        pltpu.SemaphoreType.DMA,
    ],
)
def _edge_kernel(y_hbm, src_hbm, dst_hbm, z_hbm,
                 shared_z, rows0, rows1, sidx, didx, sem0, sem1):
    cid = lax.axis_index("c")
    sid = lax.axis_index("s")

    def fill(i, _):
        for j in range(D // L):
            rows0[i, pl.ds(j * L, L)] = jnp.zeros((L,), jnp.float32)
        return 0

    lax.fori_loop(0, CH, fill, 0)
    for j in range(RPS // CH):
        pltpu.sync_copy(rows0, shared_z.at[pl.ds(sid * RPS + j * CH, CH)])
    plsc.subcore_barrier()

    is_fast = cid == FAST
    base_chunk = jnp.where(is_fast, sid * KF, NS * KF + sid * KS)
    n_phases = jnp.where(is_fast, NPHF, KS // PK)

    for p in range(NPHF):
        @pl.when(p < n_phases)
        def _run_phase():
            pltpu.sync_copy(src_hbm.at[pl.ds(base_chunk + p * PK, PK)], sidx)
            pltpu.sync_copy(dst_hbm.at[pl.ds(base_chunk + p * PK, PK)], didx)
            pltpu.async_copy(y_hbm.at[sidx.at[0]], rows0, sem0)
            pltpu.async_copy(y_hbm.at[sidx.at[1]], rows1, sem1)

            def step(k2, _):
                k = k2 * 2
                pltpu.make_async_copy(y_hbm.at[sidx.at[k]], rows0,
                                      sem0).wait()
                pltpu.sync_copy(rows0, shared_z.at[didx.at[k]], add=True)
                pltpu.async_copy(y_hbm.at[sidx.at[k + 2]], rows0, sem0)
                pltpu.make_async_copy(y_hbm.at[sidx.at[k + 1]], rows1,
                                      sem1).wait()
                pltpu.sync_copy(rows1, shared_z.at[didx.at[k + 1]],
                                add=True)
                pltpu.async_copy(y_hbm.at[sidx.at[k + 3]], rows1, sem1)
                return 0

            lax.fori_loop(0, PK2 - 1, step, 0)
            pltpu.make_async_copy(y_hbm.at[sidx.at[PK - 2]], rows0,
                                  sem0).wait()
            pltpu.sync_copy(rows0, shared_z.at[didx.at[PK - 2]], add=True)
            pltpu.make_async_copy(y_hbm.at[sidx.at[PK - 1]], rows1,
                                  sem1).wait()
            pltpu.sync_copy(rows1, shared_z.at[didx.at[PK - 1]], add=True)

    plsc.subcore_barrier()
    pltpu.sync_copy(
        shared_z.at[pl.ds(sid * RPS, RPS)],
        z_hbm.at[cid, pl.ds(sid * RPS, RPS)],
    )


BR = 1024


def _mm1_body(hist_ref, x_ref, w_ref, y_ref, dinv_ref):
    deg = hist_ref[0][:, 0:1] + hist_ref[1][:, 0:1] + 1.0
    dinv = lax.rsqrt(deg)
    xw = jnp.dot(x_ref[...], w_ref[...], preferred_element_type=jnp.float32)
    y_ref[...] = xw * dinv
    dinv_ref[...] = jnp.broadcast_to(dinv, (BR, D))


def _mm1(hist, xp, W1):
    return pl.pallas_call(
        _mm1_body,
        grid=(NPAD // BR,),
        in_specs=[
            pl.BlockSpec((NC, BR, D), lambda i: (0, i, 0)),
            pl.BlockSpec((BR, D), lambda i: (i, 0)),
            pl.BlockSpec((D, D), lambda i: (0, 0)),
        ],
        out_specs=[
            pl.BlockSpec((BR, D), lambda i: (i, 0)),
            pl.BlockSpec((BR, D), lambda i: (i, 0)),
        ],
        out_shape=[
            jax.ShapeDtypeStruct((NPAD, D), jnp.float32),
            jax.ShapeDtypeStruct((NPAD, D), jnp.float32),
        ],
    )(hist, xp, W1)


def _mm2_body(z_ref, y_ref, dinv_ref, b_ref, w_ref, o_ref):
    h = (z_ref[0] + z_ref[1] + y_ref[...]) * dinv_ref[...] + b_ref[...]
    h = jnp.maximum(h, 0.0)
    o_ref[...] = (
        jnp.dot(h, w_ref[...], preferred_element_type=jnp.float32)
        * dinv_ref[...]
    )


def _mm2(z, y, dinv, b, W2):
    return pl.pallas_call(
        _mm2_body,
        grid=(NPAD // BR,),
        in_specs=[
            pl.BlockSpec((NC, BR, D), lambda i: (0, i, 0)),
            pl.BlockSpec((BR, D), lambda i: (i, 0)),
            pl.BlockSpec((BR, D), lambda i: (i, 0)),
            pl.BlockSpec((1, D), lambda i: (0, 0)),
            pl.BlockSpec((D, D), lambda i: (0, 0)),
        ],
        out_specs=pl.BlockSpec((BR, D), lambda i: (i, 0)),
        out_shape=jax.ShapeDtypeStruct((NPAD, D), jnp.float32),
    )(z, y, dinv, b, W2)


def _fin_body(z_ref, y_ref, dinv_ref, b_ref, o_ref):
    o_ref[...] = (z_ref[0] + z_ref[1] + y_ref[...]) * dinv_ref[...] + b_ref[...]


def _fin(z, y, dinv, b):
    return pl.pallas_call(
        _fin_body,
        grid=(NPAD // BR,),
        in_specs=[
            pl.BlockSpec((NC, BR, D), lambda i: (0, i, 0)),
            pl.BlockSpec((BR, D), lambda i: (i, 0)),
            pl.BlockSpec((BR, D), lambda i: (i, 0)),
            pl.BlockSpec((1, D), lambda i: (0, 0)),
        ],
        out_specs=pl.BlockSpec((BR, D), lambda i: (i, 0)),
        out_shape=jax.ShapeDtypeStruct((NPAD, D), jnp.float32),
    )(z, y, dinv, b)


def kernel(x, edge_index, W1, b1, W2, b2):
    ei = edge_index.astype(jnp.int32)
    src = jnp.concatenate([ei[0], jnp.zeros((EPAD - E,), jnp.int32)])
    dst = jnp.concatenate([ei[1], jnp.full((EPAD - E,), N, jnp.int32)])
    src2d = src.reshape(NCHUNK, CH)
    dst2d = dst.reshape(NCHUNK, CH)
    xp = jnp.concatenate([x, jnp.zeros((NPAD - N, D), jnp.float32)])

    hist = _deg_kernel(dst2d)
    y1, dinv = _mm1(hist, xp, W1)
    z1 = _edge_kernel(y1, src2d, dst2d)
    y2 = _mm2(z1, y1, dinv, b1.reshape(1, D), W2)
    z2 = _edge_kernel(y2, src2d, dst2d)
    out = _fin(z2, y2, dinv, b2.reshape(1, D))
    return out[:N]


# speedup vs baseline: 3.2880x; 3.2880x over previous
import functools

import jax
import jax.numpy as jnp
from jax import lax
from jax.experimental import pallas as pl
from jax.experimental.pallas import tpu as pltpu
from jax.experimental.pallas import tpu_sc as plsc

N = 10000
D = 128
NPAD = 10240
E = 320000
NC, NS, L = 2, 16, 16
NW = NC * NS
EPT = 10240
EPAD = EPT * NW
RPS = NPAD // NS
CH = 128
K = EPT // CH
NCHUNK = NW * K
PK = 40
PK2 = PK // 2
NPHF = K // PK

_mesh = plsc.VectorSubcoreMesh(core_axis_name="c", subcore_axis_name="s")


@functools.partial(
    pl.kernel,
    out_type=jax.ShapeDtypeStruct((NC, NPAD, D), jnp.float32),
    mesh=_mesh,
    scratch_types=[
        pltpu.VMEM_SHARED((NPAD, D), jnp.float32),
        pltpu.VMEM((CH, D), jnp.float32),
        pltpu.VMEM((K, CH), jnp.int32),
        pltpu.VMEM((64, D), jnp.float32),
        pltpu.SemaphoreType.DMA,
    ],
)
def _deg_kernel(dst_hbm, hist_hbm, shared_h, ones_v, didx, zbuf, sem):
    cid = lax.axis_index("c")
    sid = lax.axis_index("s")
    wid = sid * NC + cid

    def fill(i, _):
        for j in range(D // L):
            zbuf[i, pl.ds(j * L, L)] = jnp.zeros((L,), jnp.float32)
        return 0

    lax.fori_loop(0, 64, fill, 0)

    def fill1(i, _):
        for j in range(D // L):
            ones_v[i, pl.ds(j * L, L)] = jnp.ones((L,), jnp.float32)
        return 0

    lax.fori_loop(0, CH, fill1, 0)
    for j in range(RPS // 64):
        pltpu.sync_copy(zbuf, shared_h.at[pl.ds(sid * RPS + j * 64, 64)])
    pltpu.sync_copy(dst_hbm.at[pl.ds(wid * K, K)], didx)
    plsc.subcore_barrier()

    def grp(g, _):
        k = g * 8
        for j in range(8):
            pltpu.async_copy(ones_v, shared_h.at[didx.at[k + j]], sem,
                             add=True)
        for j in range(8):
            pltpu.make_async_copy(ones_v, shared_h.at[didx.at[k + j]],
                                  sem).wait()
        return 0

    lax.fori_loop(0, K // 8, grp, 0)
    plsc.subcore_barrier()
    pltpu.sync_copy(
        shared_h.at[pl.ds(sid * RPS, RPS)],
        hist_hbm.at[cid, pl.ds(sid * RPS, RPS)],
    )


@functools.partial(
    pl.kernel,
    out_type=jax.ShapeDtypeStruct((NC, NPAD, D), jnp.float32),
    mesh=_mesh,
    scratch_types=[
        pltpu.VMEM_SHARED((NPAD, D), jnp.float32),
        pltpu.VMEM((CH, D), jnp.float32),
        pltpu.VMEM((CH, D), jnp.float32),
        pltpu.VMEM((PK, CH), jnp.int32),
        pltpu.VMEM((PK, CH), jnp.int32),
        pltpu.SemaphoreType.DMA,
        pltpu.SemaphoreType.DMA,
    ],
)
def _edge_kernel(y_hbm, src_hbm, dst_hbm, z_hbm,
                 shared_z, rows0, rows1, sidx, didx, sem0, sem1):
    cid = lax.axis_index("c")
    sid = lax.axis_index("s")

    def fill(i, _):
        for j in range(D // L):
            rows0[i, pl.ds(j * L, L)] = jnp.zeros((L,), jnp.float32)
        return 0

    lax.fori_loop(0, CH, fill, 0)
    for j in range(RPS // CH):
        pltpu.sync_copy(rows0, shared_z.at[pl.ds(sid * RPS + j * CH, CH)])
    plsc.subcore_barrier()

    wid = sid * NC + cid
    base_chunk = wid * K

    for p in range(NPHF):
        pltpu.sync_copy(src_hbm.at[pl.ds(base_chunk + p * PK, PK)], sidx)
        pltpu.sync_copy(dst_hbm.at[pl.ds(base_chunk + p * PK, PK)], didx)
        pltpu.async_copy(y_hbm.at[sidx.at[0]], rows0, sem0)
        pltpu.async_copy(y_hbm.at[sidx.at[1]], rows1, sem1)

        def step(k2, _):
            k = k2 * 2
            pltpu.make_async_copy(y_hbm.at[sidx.at[k]], rows0,
                                  sem0).wait()
            pltpu.sync_copy(rows0, shared_z.at[didx.at[k]], add=True)
            pltpu.async_copy(y_hbm.at[sidx.at[k + 2]], rows0, sem0)
            pltpu.make_async_copy(y_hbm.at[sidx.at[k + 1]], rows1,
                                  sem1).wait()
            pltpu.sync_copy(rows1, shared_z.at[didx.at[k + 1]],
                            add=True)
            pltpu.async_copy(y_hbm.at[sidx.at[k + 3]], rows1, sem1)
            return 0

        lax.fori_loop(0, PK2 - 1, step, 0)
        pltpu.make_async_copy(y_hbm.at[sidx.at[PK - 2]], rows0,
                              sem0).wait()
        pltpu.sync_copy(rows0, shared_z.at[didx.at[PK - 2]], add=True)
        pltpu.make_async_copy(y_hbm.at[sidx.at[PK - 1]], rows1,
                              sem1).wait()
        pltpu.sync_copy(rows1, shared_z.at[didx.at[PK - 1]], add=True)

    plsc.subcore_barrier()
    pltpu.sync_copy(
        shared_z.at[pl.ds(sid * RPS, RPS)],
        z_hbm.at[cid, pl.ds(sid * RPS, RPS)],
    )


BR = 1024


def _mm1_body(hist_ref, x_ref, w_ref, y_ref, dinv_ref):
    deg = hist_ref[0][:, 0:1] + hist_ref[1][:, 0:1] + 1.0
    dinv = lax.rsqrt(deg)
    xw = jnp.dot(x_ref[...], w_ref[...], preferred_element_type=jnp.float32)
    y_ref[...] = xw * dinv
    dinv_ref[...] = jnp.broadcast_to(dinv, (BR, D))


def _mm1(hist, xp, W1):
    return pl.pallas_call(
        _mm1_body,
        grid=(NPAD // BR,),
        in_specs=[
            pl.BlockSpec((NC, BR, D), lambda i: (0, i, 0)),
            pl.BlockSpec((BR, D), lambda i: (i, 0)),
            pl.BlockSpec((D, D), lambda i: (0, 0)),
        ],
        out_specs=[
            pl.BlockSpec((BR, D), lambda i: (i, 0)),
            pl.BlockSpec((BR, D), lambda i: (i, 0)),
        ],
        out_shape=[
            jax.ShapeDtypeStruct((NPAD, D), jnp.float32),
            jax.ShapeDtypeStruct((NPAD, D), jnp.float32),
        ],
    )(hist, xp, W1)


def _mm2_body(z_ref, y_ref, dinv_ref, b_ref, w_ref, o_ref):
    h = (z_ref[0] + z_ref[1] + y_ref[...]) * dinv_ref[...] + b_ref[...]
    h = jnp.maximum(h, 0.0)
    o_ref[...] = (
        jnp.dot(h, w_ref[...], preferred_element_type=jnp.float32)
        * dinv_ref[...]
    )


def _mm2(z, y, dinv, b, W2):
    return pl.pallas_call(
        _mm2_body,
        grid=(NPAD // BR,),
        in_specs=[
            pl.BlockSpec((NC, BR, D), lambda i: (0, i, 0)),
            pl.BlockSpec((BR, D), lambda i: (i, 0)),
            pl.BlockSpec((BR, D), lambda i: (i, 0)),
            pl.BlockSpec((1, D), lambda i: (0, 0)),
            pl.BlockSpec((D, D), lambda i: (0, 0)),
        ],
        out_specs=pl.BlockSpec((BR, D), lambda i: (i, 0)),
        out_shape=jax.ShapeDtypeStruct((NPAD, D), jnp.float32),
    )(z, y, dinv, b, W2)


def _fin_body(z_ref, y_ref, dinv_ref, b_ref, o_ref):
    o_ref[...] = (z_ref[0] + z_ref[1] + y_ref[...]) * dinv_ref[...] + b_ref[...]


def _fin(z, y, dinv, b):
    return pl.pallas_call(
        _fin_body,
        grid=(NPAD // BR,),
        in_specs=[
            pl.BlockSpec((NC, BR, D), lambda i: (0, i, 0)),
            pl.BlockSpec((BR, D), lambda i: (i, 0)),
            pl.BlockSpec((BR, D), lambda i: (i, 0)),
            pl.BlockSpec((1, D), lambda i: (0, 0)),
        ],
        out_specs=pl.BlockSpec((BR, D), lambda i: (i, 0)),
        out_shape=jax.ShapeDtypeStruct((NPAD, D), jnp.float32),
    )(z, y, dinv, b)


def kernel(x, edge_index, W1, b1, W2, b2):
    ei = edge_index.astype(jnp.int32)
    pad = jnp.arange(EPAD - E, dtype=jnp.int32)
    src = jnp.concatenate([ei[0], pad % N])
    dst = jnp.concatenate([ei[1], N + pad % (NPAD - N)])
    src2d = src.reshape(NCHUNK, CH)
    dst2d = dst.reshape(NCHUNK, CH)
    xp = jnp.concatenate([x, jnp.zeros((NPAD - N, D), jnp.float32)])

    hist = _deg_kernel(dst2d)
    y1, dinv = _mm1(hist, xp, W1)
    z1 = _edge_kernel(y1, src2d, dst2d)
    y2 = _mm2(z1, y1, dinv, b1.reshape(1, D), W2)
    z2 = _edge_kernel(y2, src2d, dst2d)
    out = _fin(z2, y2, dinv, b2.reshape(1, D))
    return out[:N]
